# baseline (speedup 1.0000x reference)
import functools

import jax
import jax.numpy as jnp
from jax import lax
from jax.experimental import pallas as pl
from jax.experimental.pallas import tpu as pltpu
from jax.experimental.pallas import tpu_sc as plsc

_B = 8
_N = 8192
_S = 512
_R0SQ = 0.2 ** 2
_R1SQ = 0.4 ** 2
_K0 = 16
_K1 = 32
_NC = 2
_NS = 16
_NW = _NC * _NS
_SPW = _S // (_NW // _B)


def _fps_body(pos_ref, np_ref, dist_ref):
    px = pos_ref[0]
    py = pos_ref[1]
    pz = pos_ref[2]
    iota = lax.broadcasted_iota(jnp.int32, (_B, _N), 1)
    dist_ref[...] = jnp.full((_B, _N), 1e10, jnp.float32)

    def body(i, far):
        mask = (iota == far).astype(jnp.float32)
        cx = jnp.sum(px * mask, axis=1, keepdims=True)
        cy = jnp.sum(py * mask, axis=1, keepdims=True)
        cz = jnp.sum(pz * mask, axis=1, keepdims=True)
        np_ref[pl.ds(i, 1)] = jnp.concatenate([cx, cy, cz], axis=1)[None]
        d = (px - cx) ** 2 + (py - cy) ** 2 + (pz - cz) ** 2
        dist = jnp.minimum(dist_ref[...], d)
        dist_ref[...] = dist
        m = jnp.max(dist, axis=1, keepdims=True)
        return jnp.min(jnp.where(dist == m, iota, _N), axis=1, keepdims=True)

    lax.fori_loop(0, _S, body, jnp.zeros((_B, 1), jnp.int32))


_fps_call = pl.pallas_call(
    _fps_body,
    out_shape=jax.ShapeDtypeStruct((_S, _B, 3), jnp.float32),
    scratch_shapes=[pltpu.VMEM((_B, _N), jnp.float32)],
)


def _round_bf16(x):
    bits = plsc.bitcast(x, jnp.uint32)
    lsb = lax.shift_right_logical(bits, jnp.uint32(16)) & jnp.uint32(1)
    r = (bits + jnp.uint32(0x7FFF) + lsb) & jnp.uint32(0xFFFF0000)
    return plsc.bitcast(r, jnp.float32)


def _sc_ballquery(gpf_hbm, npos_hbm, g0_hbm, g1_hbm,
                  pts_v, cen_v, idx0_v, idx1_v, out0_v, out1_v,
                  pxr_v, pyr_v, pzr_v, psq_v):
    cid = lax.axis_index("c")
    sid = lax.axis_index("s")
    wid = sid * _NC + cid
    b = wid // (_NW // _B)
    q = wid % (_NW // _B)
    pltpu.sync_copy(gpf_hbm.at[b], pts_v)
    pltpu.sync_copy(npos_hbm.at[b, q], cen_v)

    lanei = lax.iota(jnp.int32, 16)
    zi = jnp.zeros((16,), jnp.int32)

    def pre_body(n, carry):
        for u in range(4):
            base = n * 64 + u * 16
            px = pts_v[pl.ds(3 * _N + base, 16)]
            py = pts_v[pl.ds(4 * _N + base, 16)]
            pz = pts_v[pl.ds(5 * _N + base, 16)]
            pxr_v[pl.ds(base, 16)] = _round_bf16(px)
            pyr_v[pl.ds(base, 16)] = _round_bf16(py)
            pzr_v[pl.ds(base, 16)] = _round_bf16(pz)
            psq_v[pl.ds(base, 16)] = (px * px + py * py) + pz * pz
        return carry

    lax.fori_loop(0, _N // 64, pre_body, jnp.int32(0))

    def row_body(s_local, carry):
        sv = zi + s_local
        cxv = plsc.load_gather(cen_v, [sv])
        cyv = plsc.load_gather(cen_v, [sv + _SPW])
        czv = plsc.load_gather(cen_v, [sv + 2 * _SPW])
        cxb = _round_bf16(cxv)
        cyb = _round_bf16(cyv)
        czb = _round_bf16(czv)
        csq = (cxv * cxv + cyv * cyv) + czv * czv

        def cond(st):
            n, c0, c1 = st
            return (n < _N // 32) & ((c0 < _K0) | (c1 < _K1))

        def wbody(st):
            n, c0, c1 = st
            for u in range(2):
                base = n * 32 + u * 16
                pxr = pxr_v[pl.ds(base, 16)]
                pyr = pyr_v[pl.ds(base, 16)]
                pzr = pzr_v[pl.ds(base, 16)]
                psq = psq_v[pl.ds(base, 16)]
                m = (cxb * pxr + cyb * pyr) + czb * pzr
                d2 = (csq + psq) - 2.0 * m
                lanes = lanei + base
                m0 = (d2 <= _R0SQ) & ((zi + c0) < _K0)
                m1 = (d2 <= _R1SQ) & ((zi + c1) < _K1)
                p0 = (zi + c0) + lax.cumsum(m0.astype(jnp.int32)) - 1
                p1 = (zi + c1) + lax.cumsum(m1.astype(jnp.int32)) - 1
                plsc.store_scatter(idx0_v, [p0], lanes, mask=m0)
                plsc.store_scatter(idx1_v, [p1], lanes, mask=m1)
                c0 = c0 + jnp.sum(m0.astype(jnp.int32))
                c1 = c1 + jnp.sum(m1.astype(jnp.int32))
            return (n + 1, c0, c1)

        _, cnt0, cnt1 = lax.while_loop(
            cond, wbody, (jnp.int32(0), jnp.int32(0), jnp.int32(0)))

        vec0 = idx0_v[pl.ds(0, 16)]
        first0 = plsc.load_gather(idx0_v, [zi])
        fin0 = jnp.where(lanei < (zi + cnt0), vec0, first0)
        vec1a = idx1_v[pl.ds(0, 16)]
        vec1b = idx1_v[pl.ds(16, 16)]
        first1 = plsc.load_gather(idx1_v, [zi])
        fin1a = jnp.where(lanei < (zi + cnt1), vec1a, first1)
        fin1b = jnp.where((lanei + 16) < (zi + cnt1), vec1b, first1)

        cen = (cxv, cyv, czv)
        k0row = lanei * _SPW + sv
        k1arow = lanei * _SPW + sv
        k1brow = (lanei + 16) * _SPW + sv
        for c in range(6):
            off = zi + c * _N
            v0 = plsc.load_gather(pts_v, [fin0 + off])
            v1a = plsc.load_gather(pts_v, [fin1a + off])
            v1b = plsc.load_gather(pts_v, [fin1b + off])
            if c >= 3:
                v0 = v0 - cen[c - 3]
                v1a = v1a - cen[c - 3]
                v1b = v1b - cen[c - 3]
            plsc.store_scatter(out0_v, [k0row + c * (_K0 * _SPW)], v0)
            plsc.store_scatter(out1_v, [k1arow + c * (_K1 * _SPW)], v1a)
            plsc.store_scatter(out1_v, [k1brow + c * (_K1 * _SPW)], v1b)
        return carry

    lax.fori_loop(0, _SPW, row_body, jnp.int32(0))
    pltpu.sync_copy(out0_v, g0_hbm.at[b, q])
    pltpu.sync_copy(out1_v, g1_hbm.at[b, q])


_NQ = _NW // _B


@functools.cache
def _make_sc_call():
    return functools.partial(
        pl.kernel,
        out_type=[jax.ShapeDtypeStruct((_B, _NQ, 6 * _K0 * _SPW), jnp.float32),
                  jax.ShapeDtypeStruct((_B, _NQ, 6 * _K1 * _SPW), jnp.float32)],
        mesh=plsc.VectorSubcoreMesh(core_axis_name="c", subcore_axis_name="s"),
        compiler_params=pltpu.CompilerParams(needs_layout_passes=False),
        scratch_types=[
            pltpu.VMEM((6 * _N,), jnp.float32),
            pltpu.VMEM((3 * _SPW,), jnp.float32),
            pltpu.VMEM((_K0 + 16, ), jnp.int32),
            pltpu.VMEM((_K1 + 16, ), jnp.int32),
            pltpu.VMEM((6 * _K0 * _SPW,), jnp.float32),
            pltpu.VMEM((6 * _K1 * _SPW,), jnp.float32),
            pltpu.VMEM((_N,), jnp.float32),
            pltpu.VMEM((_N,), jnp.float32),
            pltpu.VMEM((_N,), jnp.float32),
            pltpu.VMEM((_N,), jnp.float32),
        ],
    )(_sc_ballquery)


def _gram_body(x_ref, gxx_ref, gx_ref):
    x = x_ref[0]
    xx = lax.dot_general(x, x, (((1,), (1,)), ((), ())),
                         preferred_element_type=jnp.float32)
    sx = jnp.sum(x, axis=1, keepdims=True)

    @pl.when(pl.program_id(0) == 0)
    def _():
        gxx_ref[...] = xx
        gx_ref[...] = sx

    @pl.when(pl.program_id(0) != 0)
    def _():
        gxx_ref[...] += xx
        gx_ref[...] += sx


def _fused_body(x_ref, gxx_ref, gx_ref, w1_ref, b1_ref, g1_ref, be1_ref,
                w2_ref, b2_ref, o_ref, st2_ref, *, count):
    w1 = w1_ref[...]
    b1 = b1_ref[...]
    mu = gx_ref[...] / count
    w1mu = jnp.dot(w1, mu, preferred_element_type=jnp.float32)
    mean1 = w1mu + b1
    t = jnp.dot(w1, gxx_ref[...], preferred_element_type=jnp.float32)
    ey2 = (jnp.sum(t * w1, axis=1, keepdims=True) / count
           + 2.0 * b1 * w1mu + b1 * b1)
    var1 = ey2 - mean1 * mean1
    scale1 = g1_ref[...] * lax.rsqrt(var1 + 1e-5)

    x = x_ref[0]
    y1 = jnp.dot(w1, x, preferred_element_type=jnp.float32) + b1
    x2 = jnp.maximum((y1 - mean1) * scale1 + be1_ref[...], 0.0)
    y2 = jnp.dot(w2_ref[...], x2, preferred_element_type=jnp.float32) + b2_ref[...]
    st = jnp.concatenate(
        [jnp.sum(y2, axis=1, keepdims=True),
         jnp.sum(y2 * y2, axis=1, keepdims=True)], axis=1)

    first = (pl.program_id(0) == 0) & (pl.program_id(1) == 0)

    @pl.when(first)
    def _():
        st2_ref[...] = st

    @pl.when(jnp.logical_not(first))
    def _():
        st2_ref[...] += st

    @pl.when(pl.program_id(1) == 0)
    def _():
        o_ref[0] = y2

    @pl.when(pl.program_id(1) != 0)
    def _():
        o_ref[0] = jnp.maximum(o_ref[0], y2)


def _final_body(y_ref, st_ref, g_ref, be_ref, o_ref, *, count):
    y = y_ref[0]
    mean = st_ref[...][:, 0:1] / count
    var = st_ref[...][:, 1:2] / count - mean * mean
    scale = g_ref[...] * lax.rsqrt(var + 1e-5)
    o_ref[0] = jnp.maximum((y - mean) * scale + be_ref[...], 0.0)


def _mlp_branch(x, k, p1, p2):
    (w1, b1, g1, be1) = p1
    (w2, b2, g2, be2) = p2
    c1, c2 = w1.shape[0], w2.shape[0]
    m = k * _S
    count = float(_B * m)
    gxx, gx = pl.pallas_call(
        _gram_body,
        grid=(_B,),
        in_specs=[pl.BlockSpec((1, 6, m), lambda i: (i, 0, 0))],
        out_specs=[pl.BlockSpec((6, 6), lambda i: (0, 0)),
                   pl.BlockSpec((6, 1), lambda i: (0, 0))],
        out_shape=[jax.ShapeDtypeStruct((6, 6), jnp.float32),
                   jax.ShapeDtypeStruct((6, 1), jnp.float32)],
    )(x)
    ymax, st2 = pl.pallas_call(
        functools.partial(_fused_body, count=count),
        grid=(_B, k),
        in_specs=[pl.BlockSpec((1, 6, _S), lambda i, j: (i, 0, j)),
                  pl.BlockSpec((6, 6), lambda i, j: (0, 0)),
                  pl.BlockSpec((6, 1), lambda i, j: (0, 0)),
                  pl.BlockSpec((c1, 6), lambda i, j: (0, 0)),
                  pl.BlockSpec((c1, 1), lambda i, j: (0, 0)),
                  pl.BlockSpec((c1, 1), lambda i, j: (0, 0)),
                  pl.BlockSpec((c1, 1), lambda i, j: (0, 0)),
                  pl.BlockSpec((c2, c1), lambda i, j: (0, 0)),
                  pl.BlockSpec((c2, 1), lambda i, j: (0, 0))],
        out_specs=[pl.BlockSpec((1, c2, _S), lambda i, j: (i, 0, 0)),
                   pl.BlockSpec((c2, 2), lambda i, j: (0, 0))],
        out_shape=[jax.ShapeDtypeStruct((_B, c2, _S), jnp.float32),
                   jax.ShapeDtypeStruct((c2, 2), jnp.float32)],
    )(x, gxx, gx, w1, b1, g1, be1, w2, b2)
    out = pl.pallas_call(
        functools.partial(_final_body, count=count),
        grid=(_B,),
        in_specs=[pl.BlockSpec((1, c2, _S), lambda i: (i, 0, 0)),
                  pl.BlockSpec((c2, 2), lambda i: (0, 0)),
                  pl.BlockSpec((c2, 1), lambda i: (0, 0)),
                  pl.BlockSpec((c2, 1), lambda i: (0, 0))],
        out_specs=pl.BlockSpec((1, c2, _S), lambda i: (i, 0, 0)),
        out_shape=jax.ShapeDtypeStruct((_B, c2, _S), jnp.float32),
    )(ymax, st2, g2, be2)
    return out


def kernel(pos, feats, W0_0, b0_0, g0_0, be0_0, W0_1, b0_1, g0_1, be0_1,
           W1_0, b1_0, g1_0, be1_0, W1_1, b1_1, g1_1, be1_1):
    pos_t = jnp.transpose(pos, (1, 0, 2))
    np_t = _fps_call(pos_t)
    new_pos = jnp.transpose(np_t, (1, 2, 0))

    gpf = jnp.concatenate([feats, pos], axis=1)
    npq = new_pos.reshape(_B, 3, _NQ, _SPW).swapaxes(1, 2).reshape(_B, _NQ, 3 * _SPW)
    g0, g1 = _make_sc_call()(gpf.reshape(_B, 6 * _N), npq)

    def _regroup(g, k):
        g = g.reshape(_B, _NQ, 6, k, _SPW).transpose(0, 2, 3, 1, 4)
        return g.reshape(_B, 6, k * _S)

    x0 = _regroup(g0, _K0)
    x1 = _regroup(g1, _K1)
    r2 = lambda a: a.reshape(-1, 1)
    f0 = _mlp_branch(x0, _K0,
                     (W0_0, r2(b0_0), r2(g0_0), r2(be0_0)),
                     (W0_1, r2(b0_1), r2(g0_1), r2(be0_1)))
    f1 = _mlp_branch(x1, _K1,
                     (W1_0, r2(b1_0), r2(g1_0), r2(be1_0)),
                     (W1_1, r2(b1_1), r2(g1_1), r2(be1_1)))
    new_feats = jnp.concatenate([f0, f1], axis=1)
    return (new_pos, new_feats)


# speedup vs baseline: 1.2733x; 1.2733x over previous
import functools

import jax
import jax.numpy as jnp
from jax import lax
from jax.experimental import pallas as pl
from jax.experimental.pallas import tpu as pltpu
from jax.experimental.pallas import tpu_sc as plsc

_B = 8
_N = 8192
_S = 512
_R0SQ = 0.2 ** 2
_R1SQ = 0.4 ** 2
_K0 = 16
_K1 = 32
_NC = 2
_NS = 16
_NW = _NC * _NS
_SPW = _S // (_NW // _B)


def _fps_body(pos_ref, np_ref, dist_ref):
    px = pos_ref[0]
    py = pos_ref[1]
    pz = pos_ref[2]
    iota = lax.broadcasted_iota(jnp.int32, (_B, _N), 1)
    dist_ref[...] = jnp.full((_B, _N), 1e10, jnp.float32)

    def body(i, far):
        mask = (iota == far).astype(jnp.float32)
        cx = jnp.sum(px * mask, axis=1, keepdims=True)
        cy = jnp.sum(py * mask, axis=1, keepdims=True)
        cz = jnp.sum(pz * mask, axis=1, keepdims=True)
        np_ref[pl.ds(i, 1)] = jnp.concatenate([cx, cy, cz], axis=1)[None]
        d = (px - cx) ** 2 + (py - cy) ** 2 + (pz - cz) ** 2
        dist = jnp.minimum(dist_ref[...], d)
        dist_ref[...] = dist
        m = jnp.max(dist, axis=1, keepdims=True)
        return jnp.min(jnp.where(dist == m, iota, _N), axis=1, keepdims=True)

    lax.fori_loop(0, _S, body, jnp.zeros((_B, 1), jnp.int32))


_fps_call = pl.pallas_call(
    _fps_body,
    out_shape=jax.ShapeDtypeStruct((_S, _B, 3), jnp.float32),
    scratch_shapes=[pltpu.VMEM((_B, _N), jnp.float32)],
)


def _round_bf16(x):
    bits = plsc.bitcast(x, jnp.uint32)
    lsb = lax.shift_right_logical(bits, jnp.uint32(16)) & jnp.uint32(1)
    r = (bits + jnp.uint32(0x7FFF) + lsb) & jnp.uint32(0xFFFF0000)
    return plsc.bitcast(r, jnp.float32)


def _sc_ballquery(gpf_hbm, npos_hbm, g0_hbm, g1_hbm,
                  pts_v, cen_v, idx0_v, idx1_v, out0_v, out1_v,
                  pxr_v, pyr_v, pzr_v, psq_v):
    cid = lax.axis_index("c")
    sid = lax.axis_index("s")
    wid = sid * _NC + cid
    b = wid // (_NW // _B)
    q = wid % (_NW // _B)
    pltpu.sync_copy(gpf_hbm.at[b], pts_v)
    pltpu.sync_copy(npos_hbm.at[b, q], cen_v)

    lanei = lax.iota(jnp.int32, 16)
    zi = jnp.zeros((16,), jnp.int32)

    def pre_body(n, carry):
        for u in range(4):
            base = n * 64 + u * 16
            px = pts_v[pl.ds(3 * _N + base, 16)]
            py = pts_v[pl.ds(4 * _N + base, 16)]
            pz = pts_v[pl.ds(5 * _N + base, 16)]
            pxr_v[pl.ds(base, 16)] = _round_bf16(px)
            pyr_v[pl.ds(base, 16)] = _round_bf16(py)
            pzr_v[pl.ds(base, 16)] = _round_bf16(pz)
            psq_v[pl.ds(base, 16)] = (px * px + py * py) + pz * pz
        return carry

    lax.fori_loop(0, _N // 64, pre_body, jnp.int32(0))

    def row_body(s_local, carry):
        sv = zi + s_local
        cxv = plsc.load_gather(cen_v, [sv])
        cyv = plsc.load_gather(cen_v, [sv + _SPW])
        czv = plsc.load_gather(cen_v, [sv + 2 * _SPW])
        cxb = _round_bf16(cxv)
        cyb = _round_bf16(cyv)
        czb = _round_bf16(czv)
        csq = (cxv * cxv + cyv * cyv) + czv * czv

        def cond(st):
            n, c0, c1 = st
            return (n < _N // 32) & ((c0 < _K0) | (c1 < _K1))

        def wbody(st):
            n, c0, c1 = st
            for u in range(2):
                base = n * 32 + u * 16
                pxr = pxr_v[pl.ds(base, 16)]
                pyr = pyr_v[pl.ds(base, 16)]
                pzr = pzr_v[pl.ds(base, 16)]
                psq = psq_v[pl.ds(base, 16)]
                m = (cxb * pxr + cyb * pyr) + czb * pzr
                d2 = (csq + psq) - 2.0 * m
                lanes = lanei + base
                m0 = (d2 <= _R0SQ) & ((zi + c0) < _K0)
                m1 = (d2 <= _R1SQ) & ((zi + c1) < _K1)
                p0 = (zi + c0) + lax.cumsum(m0.astype(jnp.int32)) - 1
                p1 = (zi + c1) + lax.cumsum(m1.astype(jnp.int32)) - 1
                plsc.store_scatter(idx0_v, [p0], lanes, mask=m0)
                plsc.store_scatter(idx1_v, [p1], lanes, mask=m1)
                c0 = c0 + jnp.sum(m0.astype(jnp.int32))
                c1 = c1 + jnp.sum(m1.astype(jnp.int32))
            return (n + 1, c0, c1)

        _, cnt0, cnt1 = lax.while_loop(
            cond, wbody, (jnp.int32(0), jnp.int32(0), jnp.int32(0)))

        vec0 = idx0_v[pl.ds(0, 16)]
        first0 = plsc.load_gather(idx0_v, [zi])
        fin0 = jnp.where(lanei < (zi + cnt0), vec0, first0)
        vec1a = idx1_v[pl.ds(0, 16)]
        vec1b = idx1_v[pl.ds(16, 16)]
        first1 = plsc.load_gather(idx1_v, [zi])
        fin1a = jnp.where(lanei < (zi + cnt1), vec1a, first1)
        fin1b = jnp.where((lanei + 16) < (zi + cnt1), vec1b, first1)

        cen = (cxv, cyv, czv)
        k0row = lanei * _SPW + sv
        k1arow = lanei * _SPW + sv
        k1brow = (lanei + 16) * _SPW + sv
        for c in range(6):
            off = zi + c * _N
            v0 = plsc.load_gather(pts_v, [fin0 + off])
            v1a = plsc.load_gather(pts_v, [fin1a + off])
            v1b = plsc.load_gather(pts_v, [fin1b + off])
            if c >= 3:
                v0 = v0 - cen[c - 3]
                v1a = v1a - cen[c - 3]
                v1b = v1b - cen[c - 3]
            plsc.store_scatter(out0_v, [k0row + c * (_K0 * _SPW)], v0)
            plsc.store_scatter(out1_v, [k1arow + c * (_K1 * _SPW)], v1a)
            plsc.store_scatter(out1_v, [k1brow + c * (_K1 * _SPW)], v1b)
        return carry

    lax.fori_loop(0, _SPW, row_body, jnp.int32(0))
    pltpu.sync_copy(out0_v, g0_hbm.at[b, q])
    pltpu.sync_copy(out1_v, g1_hbm.at[b, q])


_NQ = _NW // _B


@functools.cache
def _make_sc_call():
    return functools.partial(
        pl.kernel,
        out_type=[jax.ShapeDtypeStruct((_B, _NQ, 6 * _K0 * _SPW), jnp.float32),
                  jax.ShapeDtypeStruct((_B, _NQ, 6 * _K1 * _SPW), jnp.float32)],
        mesh=plsc.VectorSubcoreMesh(core_axis_name="c", subcore_axis_name="s"),
        compiler_params=pltpu.CompilerParams(needs_layout_passes=False),
        scratch_types=[
            pltpu.VMEM((6 * _N,), jnp.float32),
            pltpu.VMEM((3 * _SPW,), jnp.float32),
            pltpu.VMEM((_K0 + 16, ), jnp.int32),
            pltpu.VMEM((_K1 + 16, ), jnp.int32),
            pltpu.VMEM((6 * _K0 * _SPW,), jnp.float32),
            pltpu.VMEM((6 * _K1 * _SPW,), jnp.float32),
            pltpu.VMEM((_N,), jnp.float32),
            pltpu.VMEM((_N,), jnp.float32),
            pltpu.VMEM((_N,), jnp.float32),
            pltpu.VMEM((_N,), jnp.float32),
        ],
    )(_sc_ballquery)


def _mlp1_body(x_ref, w_ref, b_ref, y_ref, st_ref):
    x = x_ref[0]
    y = jnp.dot(w_ref[...], x, preferred_element_type=jnp.float32) + b_ref[...]
    y_ref[0] = y
    st = jnp.concatenate(
        [jnp.sum(y, axis=1, keepdims=True),
         jnp.sum(y * y, axis=1, keepdims=True)], axis=1)

    @pl.when(pl.program_id(0) == 0)
    def _():
        st_ref[...] = st

    @pl.when(pl.program_id(0) != 0)
    def _():
        st_ref[...] += st


def _mlp2_body(y_ref, st_ref, g_ref, be_ref, w_ref, b2_ref, y2_ref, st2_ref,
               *, count):
    y = y_ref[0]
    mean = st_ref[...][:, 0:1] / count
    var = st_ref[...][:, 1:2] / count - mean * mean
    scale = g_ref[...] * lax.rsqrt(var + 1e-5)
    x2 = jnp.maximum((y - mean) * scale + be_ref[...], 0.0)
    y2 = jnp.dot(w_ref[...], x2, preferred_element_type=jnp.float32) + b2_ref[...]
    y2_ref[0] = y2
    st = jnp.concatenate(
        [jnp.sum(y2, axis=1, keepdims=True),
         jnp.sum(y2 * y2, axis=1, keepdims=True)], axis=1)

    @pl.when(pl.program_id(0) == 0)
    def _():
        st2_ref[...] = st

    @pl.when(pl.program_id(0) != 0)
    def _():
        st2_ref[...] += st


def _mlp3_body(y_ref, st_ref, g_ref, be_ref, o_ref, *, count, k):
    y = y_ref[0]
    mean = st_ref[...][:, 0:1] / count
    var = st_ref[...][:, 1:2] / count - mean * mean
    scale = g_ref[...] * lax.rsqrt(var + 1e-5)
    z = jnp.maximum((y - mean) * scale + be_ref[...], 0.0)
    acc = z[:, 0:_S]
    for i in range(1, k):
        acc = jnp.maximum(acc, z[:, i * _S:(i + 1) * _S])
    o_ref[0] = acc


def _mlp_branch(x, k, p1, p2):
    (w1, b1, g1, be1) = p1
    (w2, b2, g2, be2) = p2
    c1, c2 = w1.shape[0], w2.shape[0]
    m = k * _S
    count = float(_B * m)
    grid = (_B,)
    y1, st1 = pl.pallas_call(
        _mlp1_body,
        grid=grid,
        in_specs=[pl.BlockSpec((1, 6, m), lambda i: (i, 0, 0)),
                  pl.BlockSpec((c1, 6), lambda i: (0, 0)),
                  pl.BlockSpec((c1, 1), lambda i: (0, 0))],
        out_specs=[pl.BlockSpec((1, c1, m), lambda i: (i, 0, 0)),
                   pl.BlockSpec((c1, 2), lambda i: (0, 0))],
        out_shape=[jax.ShapeDtypeStruct((_B, c1, m), jnp.float32),
                   jax.ShapeDtypeStruct((c1, 2), jnp.float32)],
    )(x, w1, b1)
    y2, st2 = pl.pallas_call(
        functools.partial(_mlp2_body, count=count),
        grid=grid,
        in_specs=[pl.BlockSpec((1, c1, m), lambda i: (i, 0, 0)),
                  pl.BlockSpec((c1, 2), lambda i: (0, 0)),
                  pl.BlockSpec((c1, 1), lambda i: (0, 0)),
                  pl.BlockSpec((c1, 1), lambda i: (0, 0)),
                  pl.BlockSpec((c2, c1), lambda i: (0, 0)),
                  pl.BlockSpec((c2, 1), lambda i: (0, 0))],
        out_specs=[pl.BlockSpec((1, c2, m), lambda i: (i, 0, 0)),
                   pl.BlockSpec((c2, 2), lambda i: (0, 0))],
        out_shape=[jax.ShapeDtypeStruct((_B, c2, m), jnp.float32),
                   jax.ShapeDtypeStruct((c2, 2), jnp.float32)],
    )(y1, st1, g1, be1, w2, b2)
    out = pl.pallas_call(
        functools.partial(_mlp3_body, count=count, k=k),
        grid=grid,
        in_specs=[pl.BlockSpec((1, c2, m), lambda i: (i, 0, 0)),
                  pl.BlockSpec((c2, 2), lambda i: (0, 0)),
                  pl.BlockSpec((c2, 1), lambda i: (0, 0)),
                  pl.BlockSpec((c2, 1), lambda i: (0, 0))],
        out_specs=pl.BlockSpec((1, c2, _S), lambda i: (i, 0, 0)),
        out_shape=jax.ShapeDtypeStruct((_B, c2, _S), jnp.float32),
    )(y2, st2, g2, be2)
    return out


def kernel(pos, feats, W0_0, b0_0, g0_0, be0_0, W0_1, b0_1, g0_1, be0_1,
           W1_0, b1_0, g1_0, be1_0, W1_1, b1_1, g1_1, be1_1):
    pos_t = jnp.transpose(pos, (1, 0, 2))
    np_t = _fps_call(pos_t)
    new_pos = jnp.transpose(np_t, (1, 2, 0))

    gpf = jnp.concatenate([feats, pos], axis=1)
    npq = new_pos.reshape(_B, 3, _NQ, _SPW).swapaxes(1, 2).reshape(_B, _NQ, 3 * _SPW)
    g0, g1 = _make_sc_call()(gpf.reshape(_B, 6 * _N), npq)

    def _regroup(g, k):
        g = g.reshape(_B, _NQ, 6, k, _SPW).transpose(0, 2, 3, 1, 4)
        return g.reshape(_B, 6, k * _S)

    x0 = _regroup(g0, _K0)
    x1 = _regroup(g1, _K1)
    r2 = lambda a: a.reshape(-1, 1)
    f0 = _mlp_branch(x0, _K0,
                     (W0_0, r2(b0_0), r2(g0_0), r2(be0_0)),
                     (W0_1, r2(b0_1), r2(g0_1), r2(be0_1)))
    f1 = _mlp_branch(x1, _K1,
                     (W1_0, r2(b1_0), r2(g1_0), r2(be1_0)),
                     (W1_1, r2(b1_1), r2(g1_1), r2(be1_1)))
    new_feats = jnp.concatenate([f0, f1], axis=1)
    return (new_pos, new_feats)
